# Initial kernel scaffold; baseline (speedup 1.0000x reference)
import jax
import jax.numpy as jnp
from jax.experimental import pallas as pl


def kernel(features, rois):
    raise NotImplementedError("write your pallas kernel here")


# speedup vs baseline: 7.8404x; 7.8404x over previous
import jax
import jax.numpy as jnp
from jax import lax
from jax.experimental import pallas as pl
from jax.experimental.pallas import tpu as pltpu
from jax.experimental.pallas import tpu_sc as plsc

_AH = 7
_AW = 7
_SCALE = 0.125

_B, _C, _H, _W = 4, 256, 64, 64
_N = 1000
_PTS = _N * _AH * _AW
_NC, _NS, _L = 2, 16, 16
_NWORK = _NC * _NS
_PTS_PER_W = 1536
_PTS_PAD = _NWORK * _PTS_PER_W
_CHUNK = 64
_NCHUNKS = _PTS_PER_W // _CHUNK
_GROUPS = _C // _L


def _sc_body(table, rois, out, rois_v,
             idx0, idx1, idx2, idx3, w0, w1, w2, w3,
             ul_v, ur_v, dl_v, dr_v, out_v, sem):
    wid = lax.axis_index("s") * _NC + lax.axis_index("c")
    base_pt = wid * _PTS_PER_W

    pltpu.sync_copy(rois, rois_v)

    lanes = lax.iota(jnp.int32, _L)
    zeros_i = jnp.zeros((_L,), jnp.int32)

    def compute_meta(i, carry):
        p_local = i * _L
        p = jnp.full((_L,), base_pt, jnp.int32) + p_local + lanes
        n_raw = lax.div(p, 49)
        r = p - n_raw * 49
        ph = lax.div(r, 7)
        pw = r - ph * 7
        pad_ok = p < _PTS
        n = jnp.minimum(n_raw, _N - 1)
        n5 = n * 5
        bf = plsc.load_gather(rois_v, [n5])
        x1 = plsc.load_gather(rois_v, [n5 + 1])
        y1 = plsc.load_gather(rois_v, [n5 + 2])
        x2 = plsc.load_gather(rois_v, [n5 + 3])
        y2 = plsc.load_gather(rois_v, [n5 + 4])
        sw = x1 * _SCALE
        sh = y1 * _SCALE
        roi_w = jnp.maximum(x2 * _SCALE - sw, 0.0)
        roi_h = jnp.maximum(y2 * _SCALE - sh, 0.0)
        bin_w = roi_w / (_AW - 1.0)
        bin_h = roi_h / (_AH - 1.0)
        hh = sh + ph.astype(jnp.float32) * bin_h
        ww = sw + pw.astype(jnp.float32) * bin_w
        valid = (hh >= 0.0) & (hh < _H) & (ww >= 0.0) & (ww < _W) & pad_ok
        hi = jnp.clip(hh.astype(jnp.int32), 0, _H - 2)
        wi = jnp.clip(ww.astype(jnp.int32), 0, _W - 2)
        hr = hh - hi.astype(jnp.float32)
        wr = ww - wi.astype(jnp.float32)
        vf = jnp.where(valid, 1.0, 0.0)
        w_ul = (1.0 - hr) * (1.0 - wr) * vf
        w_ur = (1.0 - hr) * wr * vf
        w_dl = hr * (1.0 - wr) * vf
        w_dr = hr * wr * vf
        bi = bf.astype(jnp.int32)
        base_idx = bi * (_H * _W) + hi * _W + wi
        ci = lax.div(i, _CHUNK // _L)
        off = (i - ci * (_CHUNK // _L)) * _L
        idx0[ci, pl.ds(off, _L)] = base_idx
        idx1[ci, pl.ds(off, _L)] = base_idx + 1
        idx2[ci, pl.ds(off, _L)] = base_idx + _W
        idx3[ci, pl.ds(off, _L)] = base_idx + _W + 1
        w0[pl.ds(p_local, _L)] = w_ul
        w1[pl.ds(p_local, _L)] = w_ur
        w2[pl.ds(p_local, _L)] = w_dl
        w3[pl.ds(p_local, _L)] = w_dr
        return carry

    lax.fori_loop(0, _PTS_PER_W // _L, compute_meta, 0)

    def do_chunk(c, carry):
        g0 = pltpu.async_copy(table.at[idx0.at[c]], ul_v, sem)
        g1 = pltpu.async_copy(table.at[idx1.at[c]], ur_v, sem)
        g2 = pltpu.async_copy(table.at[idx2.at[c]], dl_v, sem)
        g3 = pltpu.async_copy(table.at[idx3.at[c]], dr_v, sem)
        g0.wait()
        g1.wait()
        g2.wait()
        g3.wait()

        def do_point(p, cc):
            pid = c * _CHUNK + p
            pv = jnp.full((_L,), pid, jnp.int32)
            a0 = plsc.load_gather(w0, [pv])
            a1 = plsc.load_gather(w1, [pv])
            a2 = plsc.load_gather(w2, [pv])
            a3 = plsc.load_gather(w3, [pv])
            for g in range(_GROUPS):
                sl = pl.ds(g * _L, _L)
                acc = (ul_v[p, sl] * a0 + ur_v[p, sl] * a1
                       + dl_v[p, sl] * a2 + dr_v[p, sl] * a3)
                out_v[p, sl] = acc
            return cc

        lax.fori_loop(0, _CHUNK, do_point, 0)
        pltpu.sync_copy(out_v, out.at[pl.ds(base_pt + c * _CHUNK, _CHUNK)])
        return carry

    lax.fori_loop(0, _NCHUNKS, do_chunk, 0)


def _build_sc_call():
    return pl.kernel(
        _sc_body,
        out_type=jax.ShapeDtypeStruct((_PTS_PAD, _C), jnp.float32),
        mesh=plsc.VectorSubcoreMesh(core_axis_name="c", subcore_axis_name="s"),
        compiler_params=pltpu.CompilerParams(needs_layout_passes=False),
        scratch_types=[
            pltpu.VMEM((_N * 5,), jnp.float32),
            pltpu.VMEM((_NCHUNKS, _CHUNK), jnp.int32),
            pltpu.VMEM((_NCHUNKS, _CHUNK), jnp.int32),
            pltpu.VMEM((_NCHUNKS, _CHUNK), jnp.int32),
            pltpu.VMEM((_NCHUNKS, _CHUNK), jnp.int32),
            pltpu.VMEM((_PTS_PER_W,), jnp.float32),
            pltpu.VMEM((_PTS_PER_W,), jnp.float32),
            pltpu.VMEM((_PTS_PER_W,), jnp.float32),
            pltpu.VMEM((_PTS_PER_W,), jnp.float32),
            pltpu.VMEM((_CHUNK, _C), jnp.float32),
            pltpu.VMEM((_CHUNK, _C), jnp.float32),
            pltpu.VMEM((_CHUNK, _C), jnp.float32),
            pltpu.VMEM((_CHUNK, _C), jnp.float32),
            pltpu.VMEM((_CHUNK, _C), jnp.float32),
            pltpu.SemaphoreType.DMA,
        ],
    )


def kernel(features, rois):
    table = jnp.transpose(features, (0, 2, 3, 1)).reshape(_B * _H * _W, _C)
    flat = _build_sc_call()(table, rois.reshape(_N * 5))
    out = flat[:_PTS].reshape(_N, _AH * _AW, _C)
    return jnp.transpose(out, (0, 2, 1)).reshape(_N, _C, _AH, _AW)
